# baseline (speedup 1.0000x reference)
import functools
import math

import jax
import jax.numpy as jnp
import numpy as np
from jax import lax
from jax.experimental import pallas as pl
from jax.experimental.pallas import tpu as pltpu
from jax.experimental.pallas import tpu_sc as plsc

RES = 512
TWO_PI = 2.0 * math.pi

REFINE = 64
FINE = RES * REFINE
C2 = FINE / TWO_PI
MAGIC = 1.5 * 2.0**23

INV_TWO_PI = 1.0 / TWO_PI
TWO_PI_HI = float(np.float32(TWO_PI))
S3, S5, S7, S9 = -0.16664433, 0.008315025, -0.00019337327, 2.1777885e-06

L = 16
NC, NS = 2, 16
NW = NC * NS

ROWS = 4 * 8192
COLS = 2048
ROWS_W = ROWS // NW
STRIP = 8
NCHUNK = ROWS_W // STRIP

_mesh = plsc.VectorSubcoreMesh(core_axis_name="c", subcore_axis_name="s")


@functools.partial(
    pl.kernel,
    mesh=_mesh,
    out_type=jax.ShapeDtypeStruct((ROWS, COLS), jnp.float32),
    scratch_types=[
        pltpu.VMEM((RES,), jnp.float32),
        pltpu.VMEM((RES,), jnp.float32),
        pltpu.VMEM((FINE,), jnp.float32),
        pltpu.VMEM((2, STRIP, COLS), jnp.float32),
        pltpu.VMEM((2, STRIP, COLS), jnp.float32),
        pltpu.SemaphoreType.DMA,
        pltpu.SemaphoreType.DMA,
        pltpu.SemaphoreType.DMA,
        pltpu.SemaphoreType.DMA,
    ],
    compiler_params=pltpu.CompilerParams(
        needs_layout_passes=False, use_tc_tiling_on_sc=True
    ),
)
def _sin_lut_sc(
    phase_hbm, taba_hbm, tabb_hbm, out_hbm,
    taba_v, tabb_v, fine_v, in_v, out_v, isem0, isem1, osem0, osem1,
):
    wid = lax.axis_index("s") * NC + lax.axis_index("c")
    base = wid * ROWS_W
    pltpu.sync_copy(taba_hbm, taba_v)
    pltpu.sync_copy(tabb_hbm, tabb_v)

    @plsc.parallel_loop(0, FINE, step=L, unroll=4)
    def _(j0):
        j = j0 + lax.iota(jnp.int32, L)
        s = j >> 6
        f = (j & (REFINE - 1)).astype(jnp.float32) * jnp.float32(1.0 / REFINE)
        a = plsc.load_gather(taba_v, [s])
        d = plsc.load_gather(tabb_v, [s])
        fine_v[pl.ds(j0, L)] = a + f * d

    isems = (isem0, isem1)
    osems = (osem0, osem1)

    def in_slice(c):
        return phase_hbm.at[pl.ds(base + c * STRIP, STRIP), :]

    def out_slice(c):
        return out_hbm.at[pl.ds(base + c * STRIP, STRIP), :]

    pltpu.async_copy(in_slice(0), in_v.at[0], isems[0])
    pltpu.async_copy(in_slice(1), in_v.at[1], isems[1])

    def compute(b):
        @plsc.parallel_loop(0, STRIP * COLS, step=4 * L, unroll=2)
        def _(e):
            rr = e >> 11
            cc = e & (COLS - 1)
            for v in range(3):
                x = in_v[b, rr, pl.ds(cc + v * L, L)]
                y = x * jnp.float32(C2) + jnp.float32(MAGIC)
                idx = plsc.bitcast(y, jnp.int32) & (FINE - 1)
                out_v[b, rr, pl.ds(cc + v * L, L)] = plsc.load_gather(
                    fine_v, [idx]
                )
            x = in_v[b, rr, pl.ds(cc + 3 * L, L)]
            y = x * jnp.float32(INV_TWO_PI) + jnp.float32(MAGIC)
            k = y - jnp.float32(MAGIC)
            r = x - k * jnp.float32(TWO_PI_HI)
            r2 = r * r
            r4 = r2 * r2
            e0 = jnp.float32(S3) + jnp.float32(S5) * r2
            e1 = jnp.float32(S7) + jnp.float32(S9) * r2
            out_v[b, rr, pl.ds(cc + 3 * L, L)] = r + (r * r2) * (e0 + e1 * r4)

    def step(k, carry):
        for b in (0, 1):
            c = 2 * k + b
            pltpu.make_async_copy(in_slice(c), in_v.at[b], isems[b]).wait()

            @pl.when(k >= 1)
            def _():
                pltpu.make_async_copy(out_v.at[b], out_slice(c), osems[b]).wait()

            compute(b)
            pltpu.async_copy(out_v.at[b], out_slice(c), osems[b])

            @pl.when(c + 2 < NCHUNK)
            def _():
                pltpu.async_copy(in_slice(c + 2), in_v.at[b], isems[b])
        return carry

    lax.fori_loop(0, NCHUNK // 2, step, 0)
    pltpu.make_async_copy(out_v.at[0], out_slice(NCHUNK - 2), osems[0]).wait()
    pltpu.make_async_copy(out_v.at[1], out_slice(NCHUNK - 1), osems[1]).wait()


def kernel(phase, sin_table):
    tabb = jnp.roll(sin_table, -1) - sin_table
    out = _sin_lut_sc(phase.reshape(ROWS, COLS), sin_table, tabb)
    return out.reshape(phase.shape)


# speedup vs baseline: 1.0106x; 1.0106x over previous
import functools
import math

import jax
import jax.numpy as jnp
import numpy as np
from jax import lax
from jax.experimental import pallas as pl
from jax.experimental.pallas import tpu as pltpu
from jax.experimental.pallas import tpu_sc as plsc

RES = 512
TWO_PI = 2.0 * math.pi

REFINE = 64
FINE = RES * REFINE
C2 = FINE / TWO_PI
MAGIC = 1.5 * 2.0**23

L = 16
NC, NS = 2, 16
NW = NC * NS

ROWS = 4 * 8192
COLS = 2048
ROWS_W = ROWS // NW
STRIP = 8
NCHUNK = ROWS_W // STRIP

_mesh = plsc.VectorSubcoreMesh(core_axis_name="c", subcore_axis_name="s")


@functools.partial(
    pl.kernel,
    mesh=_mesh,
    out_type=jax.ShapeDtypeStruct((ROWS, COLS), jnp.float32),
    scratch_types=[
        pltpu.VMEM((RES,), jnp.float32),
        pltpu.VMEM((RES,), jnp.float32),
        pltpu.VMEM((FINE,), jnp.float32),
        pltpu.VMEM((2, STRIP, COLS), jnp.float32),
        pltpu.VMEM((2, STRIP, COLS), jnp.float32),
        pltpu.SemaphoreType.DMA,
        pltpu.SemaphoreType.DMA,
        pltpu.SemaphoreType.DMA,
        pltpu.SemaphoreType.DMA,
    ],
    compiler_params=pltpu.CompilerParams(
        needs_layout_passes=False, use_tc_tiling_on_sc=True
    ),
)
def _sin_lut_sc(
    phase_hbm, taba_hbm, tabb_hbm, out_hbm,
    taba_v, tabb_v, fine_v, in_v, out_v, isem0, isem1, osem0, osem1,
):
    wid = lax.axis_index("s") * NC + lax.axis_index("c")
    base = wid * ROWS_W
    pltpu.sync_copy(taba_hbm, taba_v)
    pltpu.sync_copy(tabb_hbm, tabb_v)

    @plsc.parallel_loop(0, FINE, step=L, unroll=4)
    def _(j0):
        j = j0 + lax.iota(jnp.int32, L)
        s = j >> 6
        f = (j & (REFINE - 1)).astype(jnp.float32) * jnp.float32(1.0 / REFINE)
        a = plsc.load_gather(taba_v, [s])
        d = plsc.load_gather(tabb_v, [s])
        fine_v[pl.ds(j0, L)] = a + f * d

    isems = (isem0, isem1)
    osems = (osem0, osem1)

    def in_slice(c):
        return phase_hbm.at[pl.ds(base + c * STRIP, STRIP), :]

    def out_slice(c):
        return out_hbm.at[pl.ds(base + c * STRIP, STRIP), :]

    pltpu.async_copy(in_slice(0), in_v.at[0], isems[0])
    pltpu.async_copy(in_slice(1), in_v.at[1], isems[1])

    def compute(b):
        @plsc.parallel_loop(0, STRIP * COLS, step=L, unroll=32)
        def _(e):
            rr = e >> 11
            cc = e & (COLS - 1)
            x = in_v[b, rr, pl.ds(cc, L)]
            y = x * jnp.float32(C2) + jnp.float32(MAGIC)
            idx = plsc.bitcast(y, jnp.int32) & (FINE - 1)
            out_v[b, rr, pl.ds(cc, L)] = plsc.load_gather(fine_v, [idx])

    def step(k, carry):
        for b in (0, 1):
            c = 2 * k + b
            pltpu.make_async_copy(in_slice(c), in_v.at[b], isems[b]).wait()

            @pl.when(k >= 1)
            def _():
                pltpu.make_async_copy(out_v.at[b], out_slice(c), osems[b]).wait()

            compute(b)
            pltpu.async_copy(out_v.at[b], out_slice(c), osems[b])

            @pl.when(c + 2 < NCHUNK)
            def _():
                pltpu.async_copy(in_slice(c + 2), in_v.at[b], isems[b])
        return carry

    lax.fori_loop(0, NCHUNK // 2, step, 0)
    pltpu.make_async_copy(out_v.at[0], out_slice(NCHUNK - 2), osems[0]).wait()
    pltpu.make_async_copy(out_v.at[1], out_slice(NCHUNK - 1), osems[1]).wait()


def kernel(phase, sin_table):
    tabb = jnp.roll(sin_table, -1) - sin_table
    out = _sin_lut_sc(phase.reshape(ROWS, COLS), sin_table, tabb)
    return out.reshape(phase.shape)
